# Initial kernel scaffold; baseline (speedup 1.0000x reference)
import jax
import jax.numpy as jnp
from jax.experimental import pallas as pl


def kernel(x_drug, x_gene, ei_tg, ei_rt, ei_gg, params):
    raise NotImplementedError("write your pallas kernel here")


# speedup vs baseline: 2.8491x; 2.8491x over previous
import functools

import jax
import jax.numpy as jnp
from jax import lax
from jax.experimental import pallas as pl
from jax.experimental.pallas import tpu as pltpu
from jax.experimental.pallas import tpu_sc as plsc

N = 50000
NPAD = 50176
E = 200000
ECH = 1024
NCH = 196
EPAD = NCH * ECH
SUB = 8
TPR = NPAD // 16
CW = 16
NC1 = 128 // CW
NC2 = 64 // CW
BLK = 512
GRD = NPAD // BLK
EPS = 1e-5
F32 = jnp.float32


def _sc_mesh():
    return plsc.VectorSubcoreMesh(core_axis_name="c", subcore_axis_name="s")


def _deg_body(ei_hbm, ones_h, zeros_h, degp_hbm, cidx_v, ones_v, acc_sh):
    cid = lax.axis_index("c")
    t = lax.axis_index("s")
    pltpu.sync_copy(ones_h, ones_v)
    half = NCH // 2
    for r in range(3):
        pltpu.sync_copy(zeros_h, acc_sh.at[pl.ds(t * TPR, TPR)])
        plsc.subcore_barrier()

        def body(k, carry):
            c = cid * half + t + 16 * k

            @pl.when(c < (cid + 1) * half)
            def _():
                pltpu.sync_copy(ei_hbm.at[r, 1, c], cidx_v)
                for j in range(SUB):
                    pltpu.sync_copy(ones_v, acc_sh.at[cidx_v.at[j]],
                                    add=True)
            return carry

        lax.fori_loop(0, 7, body, 0)
        plsc.subcore_barrier()
        pltpu.sync_copy(acc_sh.at[pl.ds(t * TPR, TPR)],
                        degp_hbm.at[cid, r, pl.ds(t * TPR, TPR)])
        plsc.subcore_barrier()


def _deg(ei_all, ones_hbm, zeros_hbm):
    k = functools.partial(
        pl.kernel,
        out_type=jax.ShapeDtypeStruct((2, 3, NPAD, CW), F32),
        mesh=_sc_mesh(),
        scratch_types=[
            pltpu.VMEM((SUB, 128), jnp.int32),
            pltpu.VMEM((128, CW), F32),
            pltpu.VMEM_SHARED((NPAD, CW), F32),
        ],
        compiler_params=pltpu.CompilerParams(use_tc_tiling_on_sc=False),
    )(_deg_body)
    return k(ei_all, ones_hbm, zeros_hbm)


def _msg_body(npass, cpr, tab_hbm, ei_hbm, zeros_h, s_hbm,
              ridx_v, cidx_v, rows_v, acc_sh, sem):
    cid = lax.axis_index("c")
    t = lax.axis_index("s")
    half = npass // 2

    def pass_body(p, carry):
        r = p // cpr
        pltpu.sync_copy(zeros_h, acc_sh.at[pl.ds(t * TPR, TPR)])
        plsc.subcore_barrier()

        def chunk_body(k, c2):
            c = t + 16 * k

            @pl.when(c < NCH)
            def _():
                pltpu.sync_copy(ei_hbm.at[r, 0, c], ridx_v)
                pltpu.sync_copy(ei_hbm.at[r, 1, c], cidx_v)
                for j in range(SUB):
                    rows = rows_v.at[pl.ds(j * 128, 128)]
                    pltpu.async_copy(tab_hbm.at[p].at[ridx_v.at[j]],
                                     rows, sem).wait()
                    pltpu.sync_copy(rows, acc_sh.at[cidx_v.at[j]], add=True)
            return c2

        lax.fori_loop(0, 13, chunk_body, 0)
        plsc.subcore_barrier()
        pltpu.sync_copy(acc_sh.at[pl.ds(t * TPR, TPR)],
                        s_hbm.at[p, pl.ds(t * TPR, TPR)])
        return carry

    lax.fori_loop(cid * half, (cid + 1) * half, pass_body, 0)


def _msg(tables, ei_all, zeros_hbm, npass, cpr):
    k = functools.partial(
        pl.kernel,
        out_type=jax.ShapeDtypeStruct((npass, NPAD, CW), F32),
        mesh=_sc_mesh(),
        scratch_types=[
            pltpu.VMEM((SUB, 128), jnp.int32),
            pltpu.VMEM((SUB, 128), jnp.int32),
            pltpu.VMEM((ECH, CW), F32),
            pltpu.VMEM_SHARED((NPAD, CW), F32),
            pltpu.SemaphoreType.DMA,
        ],
        compiler_params=pltpu.CompilerParams(use_tc_tiling_on_sc=False),
    )(functools.partial(_msg_body, npass, cpr))
    return k(tables, ei_all, zeros_hbm)


def _ln(x, g, b):
    mu = jnp.mean(x, axis=-1, keepdims=True)
    xc = x - mu
    var = jnp.mean(xc * xc, axis=-1, keepdims=True)
    return xc * lax.rsqrt(var + EPS) * g + b


def _dis_of(dp):
    deg = dp[0, :, :, 0:1] + dp[1, :, :, 0:1]
    return jnp.where(deg > 0, lax.rsqrt(jnp.maximum(deg, 1e-12)), 0.0)


def _chunks(y, n):
    return [y[:, CW * i:CW * (i + 1)] for i in range(n)]


def _pre_body(xd_ref, xg_ref, dp_ref, wind, bind, wing, bing,
              wtg, wgg, wrt, t1_ref, dis_ref):
    dis = _dis_of(dp_ref[...])
    dis_ref[...] = dis
    hd = jnp.dot(xd_ref[...], wind[...], preferred_element_type=F32) + bind[...]
    hg = jnp.dot(xg_ref[...], wing[...], preferred_element_type=F32) + bing[...]
    y_tg = jnp.dot(hd, wtg[...], preferred_element_type=F32) * dis[0]
    y_gg = jnp.dot(hg, wgg[...], preferred_element_type=F32) * dis[1]
    y_rt = jnp.dot(hg, wrt[...], preferred_element_type=F32) * dis[2]
    t1_ref[...] = jnp.stack(
        _chunks(y_tg, NC1) + _chunks(y_gg, NC1) + _chunks(y_rt, NC1), axis=0)


def _tc_pre(xd, xg, degp, wind, bind, wing, bing, wtg, wgg, wrt):
    row = pl.BlockSpec((BLK, 128), lambda i: (i, 0))
    full = pl.BlockSpec((128, 128), lambda i: (0, 0))
    vec = pl.BlockSpec((1, 128), lambda i: (0, 0))
    return pl.pallas_call(
        _pre_body,
        grid=(GRD,),
        in_specs=[row, row,
                  pl.BlockSpec((2, 3, BLK, CW), lambda i: (0, 0, i, 0)),
                  full, vec, full, vec, full, full, full],
        out_specs=[pl.BlockSpec((3 * NC1, BLK, CW), lambda i: (0, i, 0)),
                   pl.BlockSpec((3, BLK, 1), lambda i: (0, i, 0))],
        out_shape=[jax.ShapeDtypeStruct((3 * NC1, NPAD, CW), F32),
                   jax.ShapeDtypeStruct((3, NPAD, 1), F32)],
    )(xd, xg, degp, wind, bind, wing, bing, wtg, wgg, wrt)


def _comb1_body(s_ref, dis_ref, btg, bgg, brt, lng_g, lng_b, lnd_g, lnd_b,
                wtg2, wgg2, wrt2, t2_ref):
    s = s_ref[...]
    d = dis_ref[...]
    g_tg = jnp.concatenate([s[i] for i in range(NC1)], axis=1)
    g_gg = jnp.concatenate([s[NC1 + i] for i in range(NC1)], axis=1)
    g_rt = jnp.concatenate([s[2 * NC1 + i] for i in range(NC1)], axis=1)
    gene = d[0] * g_tg + btg[...] + d[1] * g_gg + bgg[...]
    gene = jax.nn.relu(_ln(gene, lng_g[...], lng_b[...]))
    drug = d[2] * g_rt + brt[...]
    drug = jax.nn.relu(_ln(drug, lnd_g[...], lnd_b[...]))
    y_tg = jnp.dot(drug, wtg2[...], preferred_element_type=F32) * d[0]
    y_gg = jnp.dot(gene, wgg2[...], preferred_element_type=F32) * d[1]
    y_rt = jnp.dot(gene, wrt2[...], preferred_element_type=F32) * d[2]
    t2_ref[...] = jnp.stack(
        _chunks(y_tg, NC2) + _chunks(y_gg, NC2) + _chunks(y_rt, NC2), axis=0)


def _tc_comb1(s1, dis, btg, bgg, brt, lng_g, lng_b, lnd_g, lnd_b,
              wtg2, wgg2, wrt2):
    vec = pl.BlockSpec((1, 128), lambda i: (0, 0))
    w2 = pl.BlockSpec((128, 64), lambda i: (0, 0))
    return pl.pallas_call(
        _comb1_body,
        grid=(GRD,),
        in_specs=[pl.BlockSpec((3 * NC1, BLK, CW), lambda i: (0, i, 0)),
                  pl.BlockSpec((3, BLK, 1), lambda i: (0, i, 0)),
                  vec, vec, vec, vec, vec, vec, vec,
                  w2, w2, w2],
        out_specs=[pl.BlockSpec((3 * NC2, BLK, CW), lambda i: (0, i, 0))],
        out_shape=[jax.ShapeDtypeStruct((3 * NC2, NPAD, CW), F32)],
    )(s1, dis, btg, bgg, brt, lng_g, lng_b, lnd_g, lnd_b, wtg2, wgg2, wrt2)[0]


def _comb2_body(s_ref, dis_ref, btg, bgg, brt, lng_g, lng_b, lnd_g, lnd_b,
                drug_ref, gene_ref):
    s = s_ref[...]
    d = dis_ref[...]
    g_tg = jnp.concatenate([s[i] for i in range(NC2)], axis=1)
    g_gg = jnp.concatenate([s[NC2 + i] for i in range(NC2)], axis=1)
    g_rt = jnp.concatenate([s[2 * NC2 + i] for i in range(NC2)], axis=1)
    gene = d[0] * g_tg + btg[...] + d[1] * g_gg + bgg[...]
    gene_ref[...] = _ln(gene, lng_g[...], lng_b[...])
    drug = d[2] * g_rt + brt[...]
    drug_ref[...] = _ln(drug, lnd_g[...], lnd_b[...])


def _tc_comb2(s2, dis, btg, bgg, brt, lng_g, lng_b, lnd_g, lnd_b):
    vec = pl.BlockSpec((1, 64), lambda i: (0, 0))
    out = pl.BlockSpec((BLK, 64), lambda i: (i, 0))
    return pl.pallas_call(
        _comb2_body,
        grid=(GRD,),
        in_specs=[pl.BlockSpec((3 * NC2, BLK, CW), lambda i: (0, i, 0)),
                  pl.BlockSpec((3, BLK, 1), lambda i: (0, i, 0)),
                  vec, vec, vec, vec, vec, vec, vec],
        out_specs=[out, out],
        out_shape=[jax.ShapeDtypeStruct((NPAD, 64), F32),
                   jax.ShapeDtypeStruct((NPAD, 64), F32)],
    )(s2, dis, btg, bgg, brt, lng_g, lng_b, lnd_g, lnd_b)


def kernel(x_drug, x_gene, ei_tg, ei_rt, ei_gg, params):
    xd = jnp.pad(x_drug, ((0, NPAD - N), (0, 0)))
    xg = jnp.pad(x_gene, ((0, NPAD - N), (0, 0)))

    def prep(ei):
        e = jnp.pad(ei, ((0, 0), (0, EPAD - E)), constant_values=NPAD - 1)
        return e.reshape(2, NCH, SUB, 128)

    ei_all = jnp.stack([prep(ei_tg), prep(ei_gg), prep(ei_rt)])
    ones_hbm = jnp.ones((128, CW), F32)
    zeros_hbm = jnp.zeros((TPR, CW), F32)

    degp = _deg(ei_all, ones_hbm, zeros_hbm)

    p = params
    l1, l2 = p["layers"][0], p["layers"][1]
    r1 = lambda v: v.reshape(1, -1)

    t1, dis = _tc_pre(xd, xg, degp,
                      p["in_drug"]["W"], r1(p["in_drug"]["b"]),
                      p["in_gene"]["W"], r1(p["in_gene"]["b"]),
                      l1["tg"]["W"], l1["gg"]["W"], l1["rt"]["W"])
    s1 = _msg(t1, ei_all, zeros_hbm, 3 * NC1, NC1)
    t2 = _tc_comb1(s1, dis,
                   r1(l1["tg"]["b"]), r1(l1["gg"]["b"]), r1(l1["rt"]["b"]),
                   r1(l1["ln_gene"]["g"]), r1(l1["ln_gene"]["b"]),
                   r1(l1["ln_drug"]["g"]), r1(l1["ln_drug"]["b"]),
                   l2["tg"]["W"], l2["gg"]["W"], l2["rt"]["W"])
    s2 = _msg(t2, ei_all, zeros_hbm, 3 * NC2, NC2)
    drug_out, gene_out = _tc_comb2(
        s2, dis,
        r1(l2["tg"]["b"]), r1(l2["gg"]["b"]), r1(l2["rt"]["b"]),
        r1(l2["ln_gene"]["g"]), r1(l2["ln_gene"]["b"]),
        r1(l2["ln_drug"]["g"]), r1(l2["ln_drug"]["b"]))
    return drug_out[:N], gene_out[:N]


# speedup vs baseline: 5.7500x; 2.0182x over previous
import functools

import jax
import jax.numpy as jnp
from jax import lax
from jax.experimental import pallas as pl
from jax.experimental.pallas import tpu as pltpu
from jax.experimental.pallas import tpu_sc as plsc

N = 50000
NPAD = 50176
E = 200000
ECH = 1024
NCH = 196
EPAD = NCH * ECH
SUB = 8
WIN = 4
TPR = NPAD // 16
CW = 32
DW = 16
NC1 = 128 // CW
NC2 = 64 // CW
BLK = 512
GRD = NPAD // BLK
EPS = 1e-5
F32 = jnp.float32


def _sc_mesh():
    return plsc.VectorSubcoreMesh(core_axis_name="c", subcore_axis_name="s")


def _deg_body(ei_hbm, ones_h, zeros_h, degp_hbm, cidx_v, ones_v, acc_sh):
    cid = lax.axis_index("c")
    t = lax.axis_index("s")
    pltpu.sync_copy(ones_h, ones_v)
    half = NCH // 2
    for r in range(3):
        pltpu.sync_copy(zeros_h, acc_sh.at[pl.ds(t * TPR, TPR)])
        plsc.subcore_barrier()

        def body(k, carry):
            c = cid * half + t + 16 * k

            @pl.when(c < (cid + 1) * half)
            def _():
                pltpu.sync_copy(ei_hbm.at[r, 1, c], cidx_v)
                for j in range(SUB):
                    pltpu.sync_copy(ones_v, acc_sh.at[cidx_v.at[j]],
                                    add=True)
            return carry

        lax.fori_loop(0, 7, body, 0)
        plsc.subcore_barrier()
        pltpu.sync_copy(acc_sh.at[pl.ds(t * TPR, TPR)],
                        degp_hbm.at[cid, r, pl.ds(t * TPR, TPR)])
        plsc.subcore_barrier()


def _deg(ei_all, ones_hbm, zeros_hbm):
    k = functools.partial(
        pl.kernel,
        out_type=jax.ShapeDtypeStruct((2, 3, NPAD, DW), F32),
        mesh=_sc_mesh(),
        scratch_types=[
            pltpu.VMEM((SUB, 128), jnp.int32),
            pltpu.VMEM((128, DW), F32),
            pltpu.VMEM_SHARED((NPAD, DW), F32),
        ],
        compiler_params=pltpu.CompilerParams(use_tc_tiling_on_sc=False),
    )(_deg_body)
    return k(ei_all, ones_hbm, zeros_hbm)


def _msg_body(npass, cpr, tab_hbm, ei_hbm, zeros_h, s_hbm,
              ridx_v, cidx_v, rows_v, acc_sh, *sems):
    cid = lax.axis_index("c")
    t = lax.axis_index("s")
    half = npass // 2

    def pass_body(p, carry):
        r = p // cpr
        pltpu.sync_copy(zeros_h, acc_sh.at[pl.ds(t * TPR, TPR)])
        plsc.subcore_barrier()

        def chunk_body(k, c2):
            c = t + 16 * k

            @pl.when(c < NCH)
            def _():
                pltpu.sync_copy(ei_hbm.at[r, 0, c], ridx_v)
                pltpu.sync_copy(ei_hbm.at[r, 1, c], cidx_v)
                cps = [None] * SUB
                for j in range(WIN):
                    cps[j] = pltpu.async_copy(
                        tab_hbm.at[p].at[ridx_v.at[j]],
                        rows_v.at[pl.ds((j % WIN) * 128, 128)], sems[j % WIN])
                for j in range(SUB):
                    cps[j].wait()
                    pltpu.sync_copy(rows_v.at[pl.ds((j % WIN) * 128, 128)],
                                    acc_sh.at[cidx_v.at[j]], add=True)
                    jn = j + WIN
                    if jn < SUB:
                        cps[jn] = pltpu.async_copy(
                            tab_hbm.at[p].at[ridx_v.at[jn]],
                            rows_v.at[pl.ds((jn % WIN) * 128, 128)],
                            sems[jn % WIN])
            return c2

        lax.fori_loop(0, 13, chunk_body, 0)
        plsc.subcore_barrier()
        pltpu.sync_copy(acc_sh.at[pl.ds(t * TPR, TPR)],
                        s_hbm.at[p, pl.ds(t * TPR, TPR)])
        return carry

    lax.fori_loop(cid * half, (cid + 1) * half, pass_body, 0)


def _msg(tables, ei_all, zeros_hbm, npass, cpr):
    k = functools.partial(
        pl.kernel,
        out_type=jax.ShapeDtypeStruct((npass, NPAD, CW), F32),
        mesh=_sc_mesh(),
        scratch_types=[
            pltpu.VMEM((SUB, 128), jnp.int32),
            pltpu.VMEM((SUB, 128), jnp.int32),
            pltpu.VMEM((WIN * 128, CW), F32),
            pltpu.VMEM_SHARED((NPAD, CW), F32),
        ] + [pltpu.SemaphoreType.DMA] * WIN,
        compiler_params=pltpu.CompilerParams(use_tc_tiling_on_sc=False),
    )(functools.partial(_msg_body, npass, cpr))
    return k(tables, ei_all, zeros_hbm)


def _ln(x, g, b):
    mu = jnp.mean(x, axis=-1, keepdims=True)
    xc = x - mu
    var = jnp.mean(xc * xc, axis=-1, keepdims=True)
    return xc * lax.rsqrt(var + EPS) * g + b


def _dis_of(dp):
    deg = dp[0, :, :, 0:1] + dp[1, :, :, 0:1]
    return jnp.where(deg > 0, lax.rsqrt(jnp.maximum(deg, 1e-12)), 0.0)


def _chunks(y, n):
    return [y[:, CW * i:CW * (i + 1)] for i in range(n)]


def _pre_body(xd_ref, xg_ref, dp_ref, wind, bind, wing, bing,
              wtg, wgg, wrt, t1_ref, dis_ref):
    dis = _dis_of(dp_ref[...])
    dis_ref[...] = dis
    hd = jnp.dot(xd_ref[...], wind[...], preferred_element_type=F32) + bind[...]
    hg = jnp.dot(xg_ref[...], wing[...], preferred_element_type=F32) + bing[...]
    y_tg = jnp.dot(hd, wtg[...], preferred_element_type=F32) * dis[0]
    y_gg = jnp.dot(hg, wgg[...], preferred_element_type=F32) * dis[1]
    y_rt = jnp.dot(hg, wrt[...], preferred_element_type=F32) * dis[2]
    t1_ref[...] = jnp.stack(
        _chunks(y_tg, NC1) + _chunks(y_gg, NC1) + _chunks(y_rt, NC1), axis=0)


def _tc_pre(xd, xg, degp, wind, bind, wing, bing, wtg, wgg, wrt):
    row = pl.BlockSpec((BLK, 128), lambda i: (i, 0))
    full = pl.BlockSpec((128, 128), lambda i: (0, 0))
    vec = pl.BlockSpec((1, 128), lambda i: (0, 0))
    return pl.pallas_call(
        _pre_body,
        grid=(GRD,),
        in_specs=[row, row,
                  pl.BlockSpec((2, 3, BLK, DW), lambda i: (0, 0, i, 0)),
                  full, vec, full, vec, full, full, full],
        out_specs=[pl.BlockSpec((3 * NC1, BLK, CW), lambda i: (0, i, 0)),
                   pl.BlockSpec((3, BLK, 1), lambda i: (0, i, 0))],
        out_shape=[jax.ShapeDtypeStruct((3 * NC1, NPAD, CW), F32),
                   jax.ShapeDtypeStruct((3, NPAD, 1), F32)],
    )(xd, xg, degp, wind, bind, wing, bing, wtg, wgg, wrt)


def _comb1_body(s_ref, dis_ref, btg, bgg, brt, lng_g, lng_b, lnd_g, lnd_b,
                wtg2, wgg2, wrt2, t2_ref):
    s = s_ref[...]
    d = dis_ref[...]
    g_tg = jnp.concatenate([s[i] for i in range(NC1)], axis=1)
    g_gg = jnp.concatenate([s[NC1 + i] for i in range(NC1)], axis=1)
    g_rt = jnp.concatenate([s[2 * NC1 + i] for i in range(NC1)], axis=1)
    gene = d[0] * g_tg + btg[...] + d[1] * g_gg + bgg[...]
    gene = jax.nn.relu(_ln(gene, lng_g[...], lng_b[...]))
    drug = d[2] * g_rt + brt[...]
    drug = jax.nn.relu(_ln(drug, lnd_g[...], lnd_b[...]))
    y_tg = jnp.dot(drug, wtg2[...], preferred_element_type=F32) * d[0]
    y_gg = jnp.dot(gene, wgg2[...], preferred_element_type=F32) * d[1]
    y_rt = jnp.dot(gene, wrt2[...], preferred_element_type=F32) * d[2]
    t2_ref[...] = jnp.stack(
        _chunks(y_tg, NC2) + _chunks(y_gg, NC2) + _chunks(y_rt, NC2), axis=0)


def _tc_comb1(s1, dis, btg, bgg, brt, lng_g, lng_b, lnd_g, lnd_b,
              wtg2, wgg2, wrt2):
    vec = pl.BlockSpec((1, 128), lambda i: (0, 0))
    w2 = pl.BlockSpec((128, 64), lambda i: (0, 0))
    return pl.pallas_call(
        _comb1_body,
        grid=(GRD,),
        in_specs=[pl.BlockSpec((3 * NC1, BLK, CW), lambda i: (0, i, 0)),
                  pl.BlockSpec((3, BLK, 1), lambda i: (0, i, 0)),
                  vec, vec, vec, vec, vec, vec, vec,
                  w2, w2, w2],
        out_specs=[pl.BlockSpec((3 * NC2, BLK, CW), lambda i: (0, i, 0))],
        out_shape=[jax.ShapeDtypeStruct((3 * NC2, NPAD, CW), F32)],
    )(s1, dis, btg, bgg, brt, lng_g, lng_b, lnd_g, lnd_b, wtg2, wgg2, wrt2)[0]


def _comb2_body(s_ref, dis_ref, btg, bgg, brt, lng_g, lng_b, lnd_g, lnd_b,
                drug_ref, gene_ref):
    s = s_ref[...]
    d = dis_ref[...]
    g_tg = jnp.concatenate([s[i] for i in range(NC2)], axis=1)
    g_gg = jnp.concatenate([s[NC2 + i] for i in range(NC2)], axis=1)
    g_rt = jnp.concatenate([s[2 * NC2 + i] for i in range(NC2)], axis=1)
    gene = d[0] * g_tg + btg[...] + d[1] * g_gg + bgg[...]
    gene_ref[...] = _ln(gene, lng_g[...], lng_b[...])
    drug = d[2] * g_rt + brt[...]
    drug_ref[...] = _ln(drug, lnd_g[...], lnd_b[...])


def _tc_comb2(s2, dis, btg, bgg, brt, lng_g, lng_b, lnd_g, lnd_b):
    vec = pl.BlockSpec((1, 64), lambda i: (0, 0))
    out = pl.BlockSpec((BLK, 64), lambda i: (i, 0))
    return pl.pallas_call(
        _comb2_body,
        grid=(GRD,),
        in_specs=[pl.BlockSpec((3 * NC2, BLK, CW), lambda i: (0, i, 0)),
                  pl.BlockSpec((3, BLK, 1), lambda i: (0, i, 0)),
                  vec, vec, vec, vec, vec, vec, vec],
        out_specs=[out, out],
        out_shape=[jax.ShapeDtypeStruct((NPAD, 64), F32),
                   jax.ShapeDtypeStruct((NPAD, 64), F32)],
    )(s2, dis, btg, bgg, brt, lng_g, lng_b, lnd_g, lnd_b)


def kernel(x_drug, x_gene, ei_tg, ei_rt, ei_gg, params):
    xd = jnp.pad(x_drug, ((0, NPAD - N), (0, 0)))
    xg = jnp.pad(x_gene, ((0, NPAD - N), (0, 0)))

    def prep(ei):
        e = jnp.pad(ei, ((0, 0), (0, EPAD - E)), constant_values=NPAD - 1)
        return e.reshape(2, NCH, SUB, 128)

    ei_all = jnp.stack([prep(ei_tg), prep(ei_gg), prep(ei_rt)])
    ones_hbm = jnp.ones((128, DW), F32)
    zeros_deg = jnp.zeros((TPR, DW), F32)
    zeros_hbm = jnp.zeros((TPR, CW), F32)

    degp = _deg(ei_all, ones_hbm, zeros_deg)

    p = params
    l1, l2 = p["layers"][0], p["layers"][1]
    r1 = lambda v: v.reshape(1, -1)

    t1, dis = _tc_pre(xd, xg, degp,
                      p["in_drug"]["W"], r1(p["in_drug"]["b"]),
                      p["in_gene"]["W"], r1(p["in_gene"]["b"]),
                      l1["tg"]["W"], l1["gg"]["W"], l1["rt"]["W"])
    s1 = _msg(t1, ei_all, zeros_hbm, 3 * NC1, NC1)
    t2 = _tc_comb1(s1, dis,
                   r1(l1["tg"]["b"]), r1(l1["gg"]["b"]), r1(l1["rt"]["b"]),
                   r1(l1["ln_gene"]["g"]), r1(l1["ln_gene"]["b"]),
                   r1(l1["ln_drug"]["g"]), r1(l1["ln_drug"]["b"]),
                   l2["tg"]["W"], l2["gg"]["W"], l2["rt"]["W"])
    s2 = _msg(t2, ei_all, zeros_hbm, 3 * NC2, NC2)
    drug_out, gene_out = _tc_comb2(
        s2, dis,
        r1(l2["tg"]["b"]), r1(l2["gg"]["b"]), r1(l2["rt"]["b"]),
        r1(l2["ln_gene"]["g"]), r1(l2["ln_gene"]["b"]),
        r1(l2["ln_drug"]["g"]), r1(l2["ln_drug"]["b"]))
    return drug_out[:N], gene_out[:N]


# speedup vs baseline: 6.4591x; 1.1233x over previous
import functools

import jax
import jax.numpy as jnp
from jax import lax
from jax.experimental import pallas as pl
from jax.experimental.pallas import tpu as pltpu
from jax.experimental.pallas import tpu_sc as plsc

N = 50000
NPAD = 50176
E = 200000
ECH = 1024
NCH = 196
EPAD = NCH * ECH
SUB = 8
WIN = 4
TPR = NPAD // 16
CW = 32
DW = 16
NC1 = 128 // CW
NC2 = 64 // CW
BLK = 512
GRD = NPAD // BLK
EPS = 1e-5
F32 = jnp.float32


def _sc_mesh():
    return plsc.VectorSubcoreMesh(core_axis_name="c", subcore_axis_name="s")


def _deg_body(ei_hbm, ones_h, zeros_h, degp_hbm, cidx_v, ones_v, acc_sh):
    cid = lax.axis_index("c")
    t = lax.axis_index("s")
    pltpu.sync_copy(ones_h, ones_v)
    half = NCH // 2
    pltpu.sync_copy(zeros_h, acc_sh.at[pl.ds(t * TPR, TPR)])
    plsc.subcore_barrier()
    for r in range(3):

        def body(k, carry):
            c = cid * half + t + 16 * k

            @pl.when(c < (cid + 1) * half)
            def _():
                pltpu.sync_copy(ei_hbm.at[r, c, 1], cidx_v)
                for j in range(SUB):
                    pltpu.sync_copy(ones_v.at[r], acc_sh.at[cidx_v.at[j]],
                                    add=True)
            return carry

        lax.fori_loop(0, 7, body, 0)
    plsc.subcore_barrier()
    pltpu.sync_copy(acc_sh.at[pl.ds(t * TPR, TPR)],
                    degp_hbm.at[cid, pl.ds(t * TPR, TPR)])


def _deg(ei_all, ones_hbm, zeros_hbm):
    k = functools.partial(
        pl.kernel,
        out_type=jax.ShapeDtypeStruct((2, NPAD, DW), F32),
        mesh=_sc_mesh(),
        scratch_types=[
            pltpu.VMEM((SUB, 128), jnp.int32),
            pltpu.VMEM((3, 128, DW), F32),
            pltpu.VMEM_SHARED((NPAD, DW), F32),
        ],
        compiler_params=pltpu.CompilerParams(use_tc_tiling_on_sc=False),
    )(_deg_body)
    return k(ei_all, ones_hbm, zeros_hbm)


KMAX = (NCH + 15) // 16


def _msg_body(npass, cpr, tab_hbm, ei_hbm, zeros_h, s_hbm,
              idx_v, rows_v, acc_sh, *sems):
    cid = lax.axis_index("c")
    t = lax.axis_index("s")
    half = npass // 2

    def pass_body(p, carry):
        r = p // cpr

        def idx_copy(k):
            return pltpu.make_async_copy(ei_hbm.at[r, t + 16 * k],
                                         idx_v.at[k % 2],
                                         sems[WIN + (k % 2)])

        def gather(k, j):
            return pltpu.async_copy(
                tab_hbm.at[p].at[idx_v.at[k % 2, 0, j]],
                rows_v.at[pl.ds((j % WIN) * 128, 128)], sems[j % WIN])

        idx_copy(0).start()
        pltpu.sync_copy(zeros_h, acc_sh.at[pl.ds(t * TPR, TPR)])
        plsc.subcore_barrier()

        for k in range(KMAX):
            c = t + 16 * k
            if k + 1 < KMAX:

                @pl.when(c + 16 < NCH)
                def _(k=k):
                    idx_copy(k + 1).start()

            @pl.when(c < NCH)
            def _(k=k):
                idx_copy(k).wait()
                cps = [None] * SUB
                for j in range(WIN):
                    cps[j] = gather(k, j)
                for j in range(SUB):
                    cps[j].wait()
                    pltpu.sync_copy(rows_v.at[pl.ds((j % WIN) * 128, 128)],
                                    acc_sh.at[idx_v.at[k % 2, 1, j]],
                                    add=True)
                    if j + WIN < SUB:
                        cps[j + WIN] = gather(k, j + WIN)

        plsc.subcore_barrier()
        pltpu.sync_copy(acc_sh.at[pl.ds(t * TPR, TPR)],
                        s_hbm.at[p, pl.ds(t * TPR, TPR)])
        return carry

    lax.fori_loop(cid * half, (cid + 1) * half, pass_body, 0)


def _msg(tables, ei_all, zeros_hbm, npass, cpr):
    k = functools.partial(
        pl.kernel,
        out_type=jax.ShapeDtypeStruct((npass, NPAD, CW), F32),
        mesh=_sc_mesh(),
        scratch_types=[
            pltpu.VMEM((2, 2, SUB, 128), jnp.int32),
            pltpu.VMEM((WIN * 128, CW), F32),
            pltpu.VMEM_SHARED((NPAD, CW), F32),
        ] + [pltpu.SemaphoreType.DMA] * (WIN + 2),
        compiler_params=pltpu.CompilerParams(use_tc_tiling_on_sc=False),
    )(functools.partial(_msg_body, npass, cpr))
    return k(tables, ei_all, zeros_hbm)


def _ln(x, g, b):
    mu = jnp.mean(x, axis=-1, keepdims=True)
    xc = x - mu
    var = jnp.mean(xc * xc, axis=-1, keepdims=True)
    return xc * lax.rsqrt(var + EPS) * g + b


def _dis_of(dp):
    deg = dp[0] + dp[1]
    dis = jnp.where(deg > 0, lax.rsqrt(jnp.maximum(deg, 1e-12)), 0.0)
    return jnp.stack([dis[:, 0:1], dis[:, 1:2], dis[:, 2:3]])


def _chunks(y, n):
    return [y[:, CW * i:CW * (i + 1)] for i in range(n)]


def _pre_body(xd_ref, xg_ref, dp_ref, wind, bind, wing, bing,
              wtg, wgg, wrt, t1_ref, dis_ref):
    dis = _dis_of(dp_ref[...])
    dis_ref[...] = dis
    hd = jnp.dot(xd_ref[...], wind[...], preferred_element_type=F32) + bind[...]
    hg = jnp.dot(xg_ref[...], wing[...], preferred_element_type=F32) + bing[...]
    y_tg = jnp.dot(hd, wtg[...], preferred_element_type=F32) * dis[0]
    y_gg = jnp.dot(hg, wgg[...], preferred_element_type=F32) * dis[1]
    y_rt = jnp.dot(hg, wrt[...], preferred_element_type=F32) * dis[2]
    t1_ref[...] = jnp.stack(
        _chunks(y_tg, NC1) + _chunks(y_gg, NC1) + _chunks(y_rt, NC1), axis=0)


def _tc_pre(xd, xg, degp, wind, bind, wing, bing, wtg, wgg, wrt):
    row = pl.BlockSpec((BLK, 128), lambda i: (i, 0))
    full = pl.BlockSpec((128, 128), lambda i: (0, 0))
    vec = pl.BlockSpec((1, 128), lambda i: (0, 0))
    return pl.pallas_call(
        _pre_body,
        grid=(GRD,),
        in_specs=[row, row,
                  pl.BlockSpec((2, BLK, DW), lambda i: (0, i, 0)),
                  full, vec, full, vec, full, full, full],
        out_specs=[pl.BlockSpec((3 * NC1, BLK, CW), lambda i: (0, i, 0)),
                   pl.BlockSpec((3, BLK, 1), lambda i: (0, i, 0))],
        out_shape=[jax.ShapeDtypeStruct((3 * NC1, NPAD, CW), F32),
                   jax.ShapeDtypeStruct((3, NPAD, 1), F32)],
    )(xd, xg, degp, wind, bind, wing, bing, wtg, wgg, wrt)


def _comb1_body(s_ref, dis_ref, btg, bgg, brt, lng_g, lng_b, lnd_g, lnd_b,
                wtg2, wgg2, wrt2, t2_ref):
    s = s_ref[...]
    d = dis_ref[...]
    g_tg = jnp.concatenate([s[i] for i in range(NC1)], axis=1)
    g_gg = jnp.concatenate([s[NC1 + i] for i in range(NC1)], axis=1)
    g_rt = jnp.concatenate([s[2 * NC1 + i] for i in range(NC1)], axis=1)
    gene = d[0] * g_tg + btg[...] + d[1] * g_gg + bgg[...]
    gene = jax.nn.relu(_ln(gene, lng_g[...], lng_b[...]))
    drug = d[2] * g_rt + brt[...]
    drug = jax.nn.relu(_ln(drug, lnd_g[...], lnd_b[...]))
    y_tg = jnp.dot(drug, wtg2[...], preferred_element_type=F32) * d[0]
    y_gg = jnp.dot(gene, wgg2[...], preferred_element_type=F32) * d[1]
    y_rt = jnp.dot(gene, wrt2[...], preferred_element_type=F32) * d[2]
    t2_ref[...] = jnp.stack(
        _chunks(y_tg, NC2) + _chunks(y_gg, NC2) + _chunks(y_rt, NC2), axis=0)


def _tc_comb1(s1, dis, btg, bgg, brt, lng_g, lng_b, lnd_g, lnd_b,
              wtg2, wgg2, wrt2):
    vec = pl.BlockSpec((1, 128), lambda i: (0, 0))
    w2 = pl.BlockSpec((128, 64), lambda i: (0, 0))
    return pl.pallas_call(
        _comb1_body,
        grid=(GRD,),
        in_specs=[pl.BlockSpec((3 * NC1, BLK, CW), lambda i: (0, i, 0)),
                  pl.BlockSpec((3, BLK, 1), lambda i: (0, i, 0)),
                  vec, vec, vec, vec, vec, vec, vec,
                  w2, w2, w2],
        out_specs=[pl.BlockSpec((3 * NC2, BLK, CW), lambda i: (0, i, 0))],
        out_shape=[jax.ShapeDtypeStruct((3 * NC2, NPAD, CW), F32)],
    )(s1, dis, btg, bgg, brt, lng_g, lng_b, lnd_g, lnd_b, wtg2, wgg2, wrt2)[0]


def _comb2_body(s_ref, dis_ref, btg, bgg, brt, lng_g, lng_b, lnd_g, lnd_b,
                drug_ref, gene_ref):
    s = s_ref[...]
    d = dis_ref[...]
    g_tg = jnp.concatenate([s[i] for i in range(NC2)], axis=1)
    g_gg = jnp.concatenate([s[NC2 + i] for i in range(NC2)], axis=1)
    g_rt = jnp.concatenate([s[2 * NC2 + i] for i in range(NC2)], axis=1)
    gene = d[0] * g_tg + btg[...] + d[1] * g_gg + bgg[...]
    gene_ref[...] = _ln(gene, lng_g[...], lng_b[...])
    drug = d[2] * g_rt + brt[...]
    drug_ref[...] = _ln(drug, lnd_g[...], lnd_b[...])


def _tc_comb2(s2, dis, btg, bgg, brt, lng_g, lng_b, lnd_g, lnd_b):
    vec = pl.BlockSpec((1, 64), lambda i: (0, 0))
    out = pl.BlockSpec((BLK, 64), lambda i: (i, 0))
    return pl.pallas_call(
        _comb2_body,
        grid=(GRD,),
        in_specs=[pl.BlockSpec((3 * NC2, BLK, CW), lambda i: (0, i, 0)),
                  pl.BlockSpec((3, BLK, 1), lambda i: (0, i, 0)),
                  vec, vec, vec, vec, vec, vec, vec],
        out_specs=[out, out],
        out_shape=[jax.ShapeDtypeStruct((NPAD, 64), F32),
                   jax.ShapeDtypeStruct((NPAD, 64), F32)],
    )(s2, dis, btg, bgg, brt, lng_g, lng_b, lnd_g, lnd_b)


def kernel(x_drug, x_gene, ei_tg, ei_rt, ei_gg, params):
    xd = jnp.pad(x_drug, ((0, NPAD - N), (0, 0)))
    xg = jnp.pad(x_gene, ((0, NPAD - N), (0, 0)))

    def prep(ei):
        e = jnp.pad(ei, ((0, 0), (0, EPAD - E)), constant_values=NPAD - 1)
        return e.reshape(2, NCH, SUB, 128).transpose(1, 0, 2, 3)

    ei_all = jnp.stack([prep(ei_tg), prep(ei_gg), prep(ei_rt)])
    oh = (jnp.arange(DW)[None, :] == jnp.arange(3)[:, None]).astype(F32)
    ones_hbm = jnp.broadcast_to(oh[:, None, :], (3, 128, DW))
    zeros_deg = jnp.zeros((TPR, DW), F32)
    zeros_hbm = jnp.zeros((TPR, CW), F32)

    degp = _deg(ei_all, ones_hbm, zeros_deg)

    p = params
    l1, l2 = p["layers"][0], p["layers"][1]
    r1 = lambda v: v.reshape(1, -1)

    t1, dis = _tc_pre(xd, xg, degp,
                      p["in_drug"]["W"], r1(p["in_drug"]["b"]),
                      p["in_gene"]["W"], r1(p["in_gene"]["b"]),
                      l1["tg"]["W"], l1["gg"]["W"], l1["rt"]["W"])
    s1 = _msg(t1, ei_all, zeros_hbm, 3 * NC1, NC1)
    t2 = _tc_comb1(s1, dis,
                   r1(l1["tg"]["b"]), r1(l1["gg"]["b"]), r1(l1["rt"]["b"]),
                   r1(l1["ln_gene"]["g"]), r1(l1["ln_gene"]["b"]),
                   r1(l1["ln_drug"]["g"]), r1(l1["ln_drug"]["b"]),
                   l2["tg"]["W"], l2["gg"]["W"], l2["rt"]["W"])
    s2 = _msg(t2, ei_all, zeros_hbm, 3 * NC2, NC2)
    drug_out, gene_out = _tc_comb2(
        s2, dis,
        r1(l2["tg"]["b"]), r1(l2["gg"]["b"]), r1(l2["rt"]["b"]),
        r1(l2["ln_gene"]["g"]), r1(l2["ln_gene"]["b"]),
        r1(l2["ln_drug"]["g"]), r1(l2["ln_drug"]["b"]))
    return drug_out[:N], gene_out[:N]


# speedup vs baseline: 9.7326x; 1.5068x over previous
import functools

import jax
import jax.numpy as jnp
from jax import lax
from jax.experimental import pallas as pl
from jax.experimental.pallas import tpu as pltpu
from jax.experimental.pallas import tpu_sc as plsc

N = 50000
NPAD = 50176
E = 200000
ECH = 1024
NCH = 196
EPAD = NCH * ECH
SUB = 8
WIN = 4
TPR = NPAD // 16
CW = 32
DW = 16
NC1 = 128 // CW
NC2 = 64 // CW
BLK = 512
GRD = NPAD // BLK
EPS = 1e-5
F32 = jnp.float32


def _sc_mesh():
    return plsc.VectorSubcoreMesh(core_axis_name="c", subcore_axis_name="s")


def _deg_body(ei_hbm, ones_h, zeros_h, degp_hbm, cidx_v, ones_v, acc_sh):
    cid = lax.axis_index("c")
    t = lax.axis_index("s")
    pltpu.sync_copy(ones_h, ones_v)
    half = NCH // 2
    pltpu.sync_copy(zeros_h, acc_sh.at[pl.ds(t * TPR, TPR)])
    plsc.subcore_barrier()
    for r in range(3):

        def body(k, carry):
            c = cid * half + t + 16 * k

            @pl.when(c < (cid + 1) * half)
            def _():
                pltpu.sync_copy(ei_hbm.at[r, c], cidx_v)
                for j in range(SUB):
                    pltpu.sync_copy(ones_v.at[r], acc_sh.at[cidx_v.at[j]],
                                    add=True)
            return carry

        lax.fori_loop(0, 7, body, 0)
    plsc.subcore_barrier()
    pltpu.sync_copy(acc_sh.at[pl.ds(t * TPR, TPR)],
                    degp_hbm.at[cid, pl.ds(t * TPR, TPR)])


def _deg(ei_all, ones_hbm, zeros_hbm):
    k = functools.partial(
        pl.kernel,
        out_type=jax.ShapeDtypeStruct((2, NPAD, DW), F32),
        mesh=_sc_mesh(),
        scratch_types=[
            pltpu.VMEM((SUB, 128), jnp.int32),
            pltpu.VMEM((3, 128, DW), F32),
            pltpu.VMEM_SHARED((NPAD, DW), F32),
        ],
        compiler_params=pltpu.CompilerParams(use_tc_tiling_on_sc=False),
    )(_deg_body)
    return k(ei_all, ones_hbm, zeros_hbm)


KMAX = (NCH + 15) // 16


def _msg_body(npass, cpr, tab_hbm, ei_hbm, zeros_h, s_hbm,
              idx_v, rows_v, acc_sh, *sems):
    cid = lax.axis_index("c")
    t = lax.axis_index("s")
    half = npass // 2

    def pass_body(p, carry):
        r = p // cpr

        def idx_copy(k):
            return pltpu.make_async_copy(ei_hbm.at[r, t + 16 * k],
                                         idx_v.at[k % 2],
                                         sems[WIN + (k % 2)])

        def gather(k, j):
            return pltpu.async_copy(
                tab_hbm.at[p].at[idx_v.at[k % 2, 0, j]],
                rows_v.at[pl.ds((j % WIN) * 128, 128)], sems[j % WIN])

        idx_copy(0).start()
        pltpu.sync_copy(zeros_h, acc_sh.at[pl.ds(t * TPR, TPR)])
        plsc.subcore_barrier()

        for k in range(KMAX):
            c = t + 16 * k
            if k + 1 < KMAX:

                @pl.when(c + 16 < NCH)
                def _(k=k):
                    idx_copy(k + 1).start()

            @pl.when(c < NCH)
            def _(k=k):
                idx_copy(k).wait()
                cps = [None] * SUB
                for j in range(WIN):
                    cps[j] = gather(k, j)
                for j in range(SUB):
                    cps[j].wait()
                    pltpu.sync_copy(rows_v.at[pl.ds((j % WIN) * 128, 128)],
                                    acc_sh.at[idx_v.at[k % 2, 1, j]],
                                    add=True)
                    if j + WIN < SUB:
                        cps[j + WIN] = gather(k, j + WIN)

        plsc.subcore_barrier()
        pltpu.sync_copy(acc_sh.at[pl.ds(t * TPR, TPR)],
                        s_hbm.at[p, pl.ds(t * TPR, TPR)])
        return carry

    lax.fori_loop(cid * half, (cid + 1) * half, pass_body, 0)


def _msg(tables, ei_all, zeros_hbm, npass, cpr):
    k = functools.partial(
        pl.kernel,
        out_type=jax.ShapeDtypeStruct((npass, NPAD, CW), F32),
        mesh=_sc_mesh(),
        scratch_types=[
            pltpu.VMEM((2, 2, SUB, 128), jnp.int32),
            pltpu.VMEM((WIN * 128, CW), F32),
            pltpu.VMEM_SHARED((NPAD, CW), F32),
        ] + [pltpu.SemaphoreType.DMA] * (WIN + 2),
        compiler_params=pltpu.CompilerParams(use_tc_tiling_on_sc=False),
    )(functools.partial(_msg_body, npass, cpr))
    return k(tables, ei_all, zeros_hbm)


def _ln(x, g, b):
    mu = jnp.mean(x, axis=-1, keepdims=True)
    xc = x - mu
    var = jnp.mean(xc * xc, axis=-1, keepdims=True)
    return xc * lax.rsqrt(var + EPS) * g + b


def _dis_of(dp):
    d = dp[0] + dp[1]
    deg = jnp.concatenate([d[:, DW * a:DW * (a + 1)] for a in range(8)],
                          axis=0)
    dis = jnp.where(deg > 0, lax.rsqrt(jnp.maximum(deg, 1e-12)), 0.0)
    return jnp.stack([dis[:, 0:1], dis[:, 1:2], dis[:, 2:3]])


PK = BLK // 4


def _pack(y, n):
    out = []
    for i in range(n):
        yc = y[:, CW * i:CW * (i + 1)]
        out.append(jnp.concatenate(
            [yc[PK * a:PK * (a + 1)] for a in range(4)], axis=1))
    return out


def _unpack(s, i):
    z = s[i]
    return jnp.concatenate(
        [z[:, CW * a:CW * (a + 1)] for a in range(4)], axis=0)


def _pre_body(xd_ref, xg_ref, dp_ref, wind, bind, wing, bing,
              wtg, wgg, wrt, t1_ref, dis_ref):
    dis = _dis_of(dp_ref[...])
    dis_ref[...] = dis
    hd = jnp.dot(xd_ref[...], wind[...], preferred_element_type=F32) + bind[...]
    hg = jnp.dot(xg_ref[...], wing[...], preferred_element_type=F32) + bing[...]
    y_tg = jnp.dot(hd, wtg[...], preferred_element_type=F32) * dis[0]
    y_gg = jnp.dot(hg, wgg[...], preferred_element_type=F32) * dis[1]
    y_rt = jnp.dot(hg, wrt[...], preferred_element_type=F32) * dis[2]
    t1_ref[...] = jnp.stack(
        _pack(y_tg, NC1) + _pack(y_gg, NC1) + _pack(y_rt, NC1), axis=0)


def _tc_pre(xd, xg, degp, wind, bind, wing, bing, wtg, wgg, wrt):
    row = pl.BlockSpec((BLK, 128), lambda i: (i, 0))
    full = pl.BlockSpec((128, 128), lambda i: (0, 0))
    vec = pl.BlockSpec((1, 128), lambda i: (0, 0))
    return pl.pallas_call(
        _pre_body,
        grid=(GRD,),
        in_specs=[row, row,
                  pl.BlockSpec((2, BLK // 8, 128), lambda i: (0, i, 0)),
                  full, vec, full, vec, full, full, full],
        out_specs=[pl.BlockSpec((3 * NC1, BLK // 4, 128), lambda i: (0, i, 0)),
                   pl.BlockSpec((3, BLK, 1), lambda i: (0, i, 0))],
        out_shape=[jax.ShapeDtypeStruct((3 * NC1, NPAD // 4, 128), F32),
                   jax.ShapeDtypeStruct((3, NPAD, 1), F32)],
    )(xd, xg, degp, wind, bind, wing, bing, wtg, wgg, wrt)


def _comb1_body(s_ref, dis_ref, btg, bgg, brt, lng_g, lng_b, lnd_g, lnd_b,
                wtg2, wgg2, wrt2, t2_ref):
    s = s_ref[...]
    d = dis_ref[...]
    g_tg = jnp.concatenate([_unpack(s, i) for i in range(NC1)], axis=1)
    g_gg = jnp.concatenate([_unpack(s, NC1 + i) for i in range(NC1)], axis=1)
    g_rt = jnp.concatenate([_unpack(s, 2 * NC1 + i) for i in range(NC1)],
                           axis=1)
    gene = d[0] * g_tg + btg[...] + d[1] * g_gg + bgg[...]
    gene = jax.nn.relu(_ln(gene, lng_g[...], lng_b[...]))
    drug = d[2] * g_rt + brt[...]
    drug = jax.nn.relu(_ln(drug, lnd_g[...], lnd_b[...]))
    y_tg = jnp.dot(drug, wtg2[...], preferred_element_type=F32) * d[0]
    y_gg = jnp.dot(gene, wgg2[...], preferred_element_type=F32) * d[1]
    y_rt = jnp.dot(gene, wrt2[...], preferred_element_type=F32) * d[2]
    t2_ref[...] = jnp.stack(
        _pack(y_tg, NC2) + _pack(y_gg, NC2) + _pack(y_rt, NC2), axis=0)


def _tc_comb1(s1, dis, btg, bgg, brt, lng_g, lng_b, lnd_g, lnd_b,
              wtg2, wgg2, wrt2):
    vec = pl.BlockSpec((1, 128), lambda i: (0, 0))
    w2 = pl.BlockSpec((128, 64), lambda i: (0, 0))
    return pl.pallas_call(
        _comb1_body,
        grid=(GRD,),
        in_specs=[pl.BlockSpec((3 * NC1, BLK // 4, 128), lambda i: (0, i, 0)),
                  pl.BlockSpec((3, BLK, 1), lambda i: (0, i, 0)),
                  vec, vec, vec, vec, vec, vec, vec,
                  w2, w2, w2],
        out_specs=[pl.BlockSpec((3 * NC2, BLK // 4, 128), lambda i: (0, i, 0))],
        out_shape=[jax.ShapeDtypeStruct((3 * NC2, NPAD // 4, 128), F32)],
    )(s1, dis, btg, bgg, brt, lng_g, lng_b, lnd_g, lnd_b, wtg2, wgg2, wrt2)[0]


def _comb2_body(s_ref, dis_ref, btg, bgg, brt, lng_g, lng_b, lnd_g, lnd_b,
                drug_ref, gene_ref):
    s = s_ref[...]
    d = dis_ref[...]
    g_tg = jnp.concatenate([_unpack(s, i) for i in range(NC2)], axis=1)
    g_gg = jnp.concatenate([_unpack(s, NC2 + i) for i in range(NC2)], axis=1)
    g_rt = jnp.concatenate([_unpack(s, 2 * NC2 + i) for i in range(NC2)],
                           axis=1)
    gene = d[0] * g_tg + btg[...] + d[1] * g_gg + bgg[...]
    gene_ref[...] = _ln(gene, lng_g[...], lng_b[...])
    drug = d[2] * g_rt + brt[...]
    drug_ref[...] = _ln(drug, lnd_g[...], lnd_b[...])


def _tc_comb2(s2, dis, btg, bgg, brt, lng_g, lng_b, lnd_g, lnd_b):
    vec = pl.BlockSpec((1, 64), lambda i: (0, 0))
    out = pl.BlockSpec((BLK, 64), lambda i: (i, 0))
    return pl.pallas_call(
        _comb2_body,
        grid=(GRD,),
        in_specs=[pl.BlockSpec((3 * NC2, BLK // 4, 128), lambda i: (0, i, 0)),
                  pl.BlockSpec((3, BLK, 1), lambda i: (0, i, 0)),
                  vec, vec, vec, vec, vec, vec, vec],
        out_specs=[out, out],
        out_shape=[jax.ShapeDtypeStruct((NPAD, 64), F32),
                   jax.ShapeDtypeStruct((NPAD, 64), F32)],
    )(s2, dis, btg, bgg, brt, lng_g, lng_b, lnd_g, lnd_b)


def kernel(x_drug, x_gene, ei_tg, ei_rt, ei_gg, params):
    xd = jnp.pad(x_drug, ((0, NPAD - N), (0, 0)))
    xg = jnp.pad(x_gene, ((0, NPAD - N), (0, 0)))

    def _perm(n):
        return (n // BLK) * BLK + 4 * (n % PK) + (n % BLK) // PK

    def _perm8(n):
        q = BLK // 8
        return (n // BLK) * BLK + 8 * (n % q) + (n % BLK) // q

    def prep(ei):
        e = jnp.pad(ei, ((0, 0), (0, EPAD - E)), constant_values=NPAD - 1)
        return _perm(e).reshape(2, NCH, SUB, 128).transpose(1, 0, 2, 3)

    def prep_deg(ei):
        e = jnp.pad(ei[1], ((0, EPAD - E),), constant_values=NPAD - 1)
        return _perm8(e).reshape(NCH, SUB, 128)

    ei_all = jnp.stack([prep(ei_tg), prep(ei_gg), prep(ei_rt)])
    eid = jnp.stack([prep_deg(ei_tg), prep_deg(ei_gg), prep_deg(ei_rt)])
    oh = (jnp.arange(DW)[None, :] == jnp.arange(3)[:, None]).astype(F32)
    ones_hbm = jnp.broadcast_to(oh[:, None, :], (3, 128, DW))
    zeros_deg = jnp.zeros((TPR, DW), F32)
    zeros_hbm = jnp.zeros((TPR, CW), F32)

    degp = _deg(eid, ones_hbm, zeros_deg).reshape(2, NPAD // 8, 128)

    p = params
    l1, l2 = p["layers"][0], p["layers"][1]
    r1 = lambda v: v.reshape(1, -1)

    t1, dis = _tc_pre(xd, xg, degp,
                      p["in_drug"]["W"], r1(p["in_drug"]["b"]),
                      p["in_gene"]["W"], r1(p["in_gene"]["b"]),
                      l1["tg"]["W"], l1["gg"]["W"], l1["rt"]["W"])
    s1 = _msg(t1.reshape(3 * NC1, NPAD, CW), ei_all, zeros_hbm, 3 * NC1, NC1)
    t2 = _tc_comb1(s1.reshape(3 * NC1, NPAD // 4, 128), dis,
                   r1(l1["tg"]["b"]), r1(l1["gg"]["b"]), r1(l1["rt"]["b"]),
                   r1(l1["ln_gene"]["g"]), r1(l1["ln_gene"]["b"]),
                   r1(l1["ln_drug"]["g"]), r1(l1["ln_drug"]["b"]),
                   l2["tg"]["W"], l2["gg"]["W"], l2["rt"]["W"])
    s2 = _msg(t2.reshape(3 * NC2, NPAD, CW), ei_all, zeros_hbm, 3 * NC2, NC2)
    drug_out, gene_out = _tc_comb2(
        s2.reshape(3 * NC2, NPAD // 4, 128), dis,
        r1(l2["tg"]["b"]), r1(l2["gg"]["b"]), r1(l2["rt"]["b"]),
        r1(l2["ln_gene"]["g"]), r1(l2["ln_gene"]["b"]),
        r1(l2["ln_drug"]["g"]), r1(l2["ln_drug"]["b"]))
    return drug_out[:N], gene_out[:N]


# speedup vs baseline: 10.5135x; 1.0802x over previous
import functools

import jax
import jax.numpy as jnp
from jax import lax
from jax.experimental import pallas as pl
from jax.experimental.pallas import tpu as pltpu
from jax.experimental.pallas import tpu_sc as plsc

N = 50000
NPAD = 50176
E = 200000
ECH = 1024
NCH = 196
EPAD = NCH * ECH
SUB = 8
WIN = 4
TPR = NPAD // 16
CW = 32
DW = 16
NC1 = 128 // CW
NC2 = 64 // CW
BLK = 512
GRD = NPAD // BLK
EPS = 1e-5
F32 = jnp.float32


def _sc_mesh():
    return plsc.VectorSubcoreMesh(core_axis_name="c", subcore_axis_name="s")


def _deg_body(ei_hbm, ones_h, zeros_h, degp_hbm, cidx_v, ones_v, acc_sh):
    cid = lax.axis_index("c")
    t = lax.axis_index("s")
    pltpu.sync_copy(ones_h, ones_v)
    half = NCH // 2
    pltpu.sync_copy(zeros_h, acc_sh.at[pl.ds(t * TPR, TPR)])
    plsc.subcore_barrier()
    for r in range(3):

        def body(k, carry):
            c = cid * half + t + 16 * k

            @pl.when(c < (cid + 1) * half)
            def _():
                pltpu.sync_copy(ei_hbm.at[r, c], cidx_v)
                for j in range(SUB):
                    pltpu.sync_copy(ones_v.at[r], acc_sh.at[cidx_v.at[j]],
                                    add=True)
            return carry

        lax.fori_loop(0, 7, body, 0)
    plsc.subcore_barrier()
    pltpu.sync_copy(acc_sh.at[pl.ds(t * TPR, TPR)],
                    degp_hbm.at[cid, pl.ds(t * TPR, TPR)])


def _deg(ei_all, ones_hbm, zeros_hbm):
    k = functools.partial(
        pl.kernel,
        out_type=jax.ShapeDtypeStruct((2, NPAD, DW), F32),
        mesh=_sc_mesh(),
        scratch_types=[
            pltpu.VMEM((SUB, 128), jnp.int32),
            pltpu.VMEM((3, 128, DW), F32),
            pltpu.VMEM_SHARED((NPAD, DW), F32),
        ],
        compiler_params=pltpu.CompilerParams(use_tc_tiling_on_sc=False),
    )(_deg_body)
    return k(ei_all, ones_hbm, zeros_hbm)


KMAX = (NCH + 15) // 16


def _msg_body(poff, pcnt, toff, cpr, tab_hbm, ei_hbm, zeros_h, s_hbm,
              idx_v, rows_v, acc_sh, *sems):
    cid = lax.axis_index("c")
    t = lax.axis_index("s")
    half = pcnt // 2

    def pass_body(p, carry):
        r = p // cpr

        def idx_copy(k):
            return pltpu.make_async_copy(ei_hbm.at[r, t + 16 * k],
                                         idx_v.at[k % 2],
                                         sems[WIN + (k % 2)])

        def gather(k, j):
            return pltpu.async_copy(
                tab_hbm.at[p - toff].at[idx_v.at[k % 2, 0, j]],
                rows_v.at[pl.ds((j % WIN) * 128, 128)], sems[j % WIN])

        idx_copy(0).start()
        pltpu.sync_copy(zeros_h, acc_sh.at[pl.ds(t * TPR, TPR)])
        plsc.subcore_barrier()

        for k in range(KMAX):
            c = t + 16 * k
            if k + 1 < KMAX:

                @pl.when(c + 16 < NCH)
                def _(k=k):
                    idx_copy(k + 1).start()

            @pl.when(c < NCH)
            def _(k=k):
                idx_copy(k).wait()
                cps = [None] * SUB
                for j in range(WIN):
                    cps[j] = gather(k, j)
                for j in range(SUB):
                    cps[j].wait()
                    pltpu.sync_copy(rows_v.at[pl.ds((j % WIN) * 128, 128)],
                                    acc_sh.at[idx_v.at[k % 2, 1, j]],
                                    add=True)
                    if j + WIN < SUB:
                        cps[j + WIN] = gather(k, j + WIN)

        plsc.subcore_barrier()
        pltpu.sync_copy(acc_sh.at[pl.ds(t * TPR, TPR)],
                        s_hbm.at[p - poff, pl.ds(t * TPR, TPR)])
        return carry

    lax.fori_loop(poff + cid * half, poff + (cid + 1) * half, pass_body, 0)


def _msg(tables, ei_all, zeros_hbm, poff, pcnt, toff, cpr):
    k = functools.partial(
        pl.kernel,
        out_type=jax.ShapeDtypeStruct((pcnt, NPAD, CW), F32),
        mesh=_sc_mesh(),
        scratch_types=[
            pltpu.VMEM((2, 2, SUB, 128), jnp.int32),
            pltpu.VMEM((WIN * 128, CW), F32),
            pltpu.VMEM_SHARED((NPAD, CW), F32),
        ] + [pltpu.SemaphoreType.DMA] * (WIN + 2),
        compiler_params=pltpu.CompilerParams(use_tc_tiling_on_sc=False),
    )(functools.partial(_msg_body, poff, pcnt, toff, cpr))
    return k(tables, ei_all, zeros_hbm)


def _ln(x, g, b):
    mu = jnp.mean(x, axis=-1, keepdims=True)
    xc = x - mu
    var = jnp.mean(xc * xc, axis=-1, keepdims=True)
    return xc * lax.rsqrt(var + EPS) * g + b


def _dis_of(dp):
    d = dp[0] + dp[1]
    deg = jnp.concatenate([d[:, DW * a:DW * (a + 1)] for a in range(8)],
                          axis=0)
    dis = jnp.where(deg > 0, lax.rsqrt(jnp.maximum(deg, 1e-12)), 0.0)
    return jnp.stack([dis[:, 0:1], dis[:, 1:2], dis[:, 2:3]])


PK = BLK // 4


def _pack(y, n):
    out = []
    for i in range(n):
        yc = y[:, CW * i:CW * (i + 1)]
        out.append(jnp.concatenate(
            [yc[PK * a:PK * (a + 1)] for a in range(4)], axis=1))
    return out


def _unpack(s, i):
    z = s[i]
    return jnp.concatenate(
        [z[:, CW * a:CW * (a + 1)] for a in range(4)], axis=0)


def _pre_body(xd_ref, xg_ref, dp_ref, wind, bind, wing, bing,
              wtg, wgg, wrt, t1_ref, dis_ref):
    dis = _dis_of(dp_ref[...])
    dis_ref[...] = dis
    hd = jnp.dot(xd_ref[...], wind[...], preferred_element_type=F32) + bind[...]
    hg = jnp.dot(xg_ref[...], wing[...], preferred_element_type=F32) + bing[...]
    y_tg = jnp.dot(hd, wtg[...], preferred_element_type=F32) * dis[0]
    y_gg = jnp.dot(hg, wgg[...], preferred_element_type=F32) * dis[1]
    y_rt = jnp.dot(hg, wrt[...], preferred_element_type=F32) * dis[2]
    t1_ref[...] = jnp.stack(
        _pack(y_tg, NC1) + _pack(y_gg, NC1) + _pack(y_rt, NC1), axis=0)


def _tc_pre(xd, xg, degp, wind, bind, wing, bing, wtg, wgg, wrt):
    row = pl.BlockSpec((BLK, 128), lambda i: (i, 0))
    full = pl.BlockSpec((128, 128), lambda i: (0, 0))
    vec = pl.BlockSpec((1, 128), lambda i: (0, 0))
    return pl.pallas_call(
        _pre_body,
        grid=(GRD,),
        in_specs=[row, row,
                  pl.BlockSpec((2, BLK // 8, 128), lambda i: (0, i, 0)),
                  full, vec, full, vec, full, full, full],
        out_specs=[pl.BlockSpec((3 * NC1, BLK // 4, 128), lambda i: (0, i, 0)),
                   pl.BlockSpec((3, BLK, 1), lambda i: (0, i, 0))],
        out_shape=[jax.ShapeDtypeStruct((3 * NC1, NPAD // 4, 128), F32),
                   jax.ShapeDtypeStruct((3, NPAD, 1), F32)],
    )(xd, xg, degp, wind, bind, wing, bing, wtg, wgg, wrt)


def _comb1g_body(s_ref, dis_ref, btg, bgg, lng_g, lng_b,
                 wgg2, wrt2, t2_ref):
    s = s_ref[...]
    d = dis_ref[...]
    g_tg = jnp.concatenate([_unpack(s, i) for i in range(NC1)], axis=1)
    g_gg = jnp.concatenate([_unpack(s, NC1 + i) for i in range(NC1)], axis=1)
    gene = d[0] * g_tg + btg[...] + d[1] * g_gg + bgg[...]
    gene = jax.nn.relu(_ln(gene, lng_g[...], lng_b[...]))
    y_gg = jnp.dot(gene, wgg2[...], preferred_element_type=F32) * d[1]
    y_rt = jnp.dot(gene, wrt2[...], preferred_element_type=F32) * d[2]
    t2_ref[...] = jnp.stack(_pack(y_gg, NC2) + _pack(y_rt, NC2), axis=0)


def _tc_comb1g(s1a, dis, btg, bgg, lng_g, lng_b, wgg2, wrt2):
    vec = pl.BlockSpec((1, 128), lambda i: (0, 0))
    w2 = pl.BlockSpec((128, 64), lambda i: (0, 0))
    return pl.pallas_call(
        _comb1g_body,
        grid=(GRD,),
        in_specs=[pl.BlockSpec((2 * NC1, BLK // 4, 128), lambda i: (0, i, 0)),
                  pl.BlockSpec((3, BLK, 1), lambda i: (0, i, 0)),
                  vec, vec, vec, vec, w2, w2],
        out_specs=[pl.BlockSpec((2 * NC2, BLK // 4, 128), lambda i: (0, i, 0))],
        out_shape=[jax.ShapeDtypeStruct((2 * NC2, NPAD // 4, 128), F32)],
    )(s1a, dis, btg, bgg, lng_g, lng_b, wgg2, wrt2)[0]


def _comb1d_body(s_ref, dis_ref, brt, lnd_g, lnd_b, wtg2, t2_ref):
    s = s_ref[...]
    d = dis_ref[...]
    g_rt = jnp.concatenate([_unpack(s, i) for i in range(NC1)], axis=1)
    drug = d[2] * g_rt + brt[...]
    drug = jax.nn.relu(_ln(drug, lnd_g[...], lnd_b[...]))
    y_tg = jnp.dot(drug, wtg2[...], preferred_element_type=F32) * d[0]
    t2_ref[...] = jnp.stack(_pack(y_tg, NC2), axis=0)


def _tc_comb1d(s1b, dis, brt, lnd_g, lnd_b, wtg2):
    vec = pl.BlockSpec((1, 128), lambda i: (0, 0))
    w2 = pl.BlockSpec((128, 64), lambda i: (0, 0))
    return pl.pallas_call(
        _comb1d_body,
        grid=(GRD,),
        in_specs=[pl.BlockSpec((NC1, BLK // 4, 128), lambda i: (0, i, 0)),
                  pl.BlockSpec((3, BLK, 1), lambda i: (0, i, 0)),
                  vec, vec, vec, w2],
        out_specs=[pl.BlockSpec((NC2, BLK // 4, 128), lambda i: (0, i, 0))],
        out_shape=[jax.ShapeDtypeStruct((NC2, NPAD // 4, 128), F32)],
    )(s1b, dis, brt, lnd_g, lnd_b, wtg2)[0]


def _comb2_body(sd_ref, sg_ref, dis_ref, btg, bgg, brt,
                lng_g, lng_b, lnd_g, lnd_b, drug_ref, gene_ref):
    sd = sd_ref[...]
    sg = sg_ref[...]
    d = dis_ref[...]
    g_tg = jnp.concatenate([_unpack(sd, i) for i in range(NC2)], axis=1)
    g_gg = jnp.concatenate([_unpack(sg, i) for i in range(NC2)], axis=1)
    g_rt = jnp.concatenate([_unpack(sg, NC2 + i) for i in range(NC2)], axis=1)
    gene = d[0] * g_tg + btg[...] + d[1] * g_gg + bgg[...]
    gene_ref[...] = _ln(gene, lng_g[...], lng_b[...])
    drug = d[2] * g_rt + brt[...]
    drug_ref[...] = _ln(drug, lnd_g[...], lnd_b[...])


def _tc_comb2(s2d, s2g, dis, btg, bgg, brt, lng_g, lng_b, lnd_g, lnd_b):
    vec = pl.BlockSpec((1, 64), lambda i: (0, 0))
    out = pl.BlockSpec((BLK, 64), lambda i: (i, 0))
    return pl.pallas_call(
        _comb2_body,
        grid=(GRD,),
        in_specs=[pl.BlockSpec((NC2, BLK // 4, 128), lambda i: (0, i, 0)),
                  pl.BlockSpec((2 * NC2, BLK // 4, 128), lambda i: (0, i, 0)),
                  pl.BlockSpec((3, BLK, 1), lambda i: (0, i, 0)),
                  vec, vec, vec, vec, vec, vec, vec],
        out_specs=[out, out],
        out_shape=[jax.ShapeDtypeStruct((NPAD, 64), F32),
                   jax.ShapeDtypeStruct((NPAD, 64), F32)],
    )(s2d, s2g, dis, btg, bgg, brt, lng_g, lng_b, lnd_g, lnd_b)


def kernel(x_drug, x_gene, ei_tg, ei_rt, ei_gg, params):
    xd = jnp.pad(x_drug, ((0, NPAD - N), (0, 0)))
    xg = jnp.pad(x_gene, ((0, NPAD - N), (0, 0)))

    def _perm(n):
        return (n // BLK) * BLK + 4 * (n % PK) + (n % BLK) // PK

    def _perm8(n):
        q = BLK // 8
        return (n // BLK) * BLK + 8 * (n % q) + (n % BLK) // q

    def prep(ei):
        e = jnp.pad(ei, ((0, 0), (0, EPAD - E)), constant_values=NPAD - 1)
        return _perm(e).reshape(2, NCH, SUB, 128).transpose(1, 0, 2, 3)

    def prep_deg(ei):
        e = jnp.pad(ei[1], ((0, EPAD - E),), constant_values=NPAD - 1)
        return _perm8(e).reshape(NCH, SUB, 128)

    ei_all = jnp.stack([prep(ei_tg), prep(ei_gg), prep(ei_rt)])
    eid = jnp.stack([prep_deg(ei_tg), prep_deg(ei_gg), prep_deg(ei_rt)])
    oh = (jnp.arange(DW)[None, :] == jnp.arange(3)[:, None]).astype(F32)
    ones_hbm = jnp.broadcast_to(oh[:, None, :], (3, 128, DW))
    zeros_deg = jnp.zeros((TPR, DW), F32)
    zeros_hbm = jnp.zeros((TPR, CW), F32)

    degp = _deg(eid, ones_hbm, zeros_deg).reshape(2, NPAD // 8, 128)

    p = params
    l1, l2 = p["layers"][0], p["layers"][1]
    r1 = lambda v: v.reshape(1, -1)

    t1, dis = _tc_pre(xd, xg, degp,
                      p["in_drug"]["W"], r1(p["in_drug"]["b"]),
                      p["in_gene"]["W"], r1(p["in_gene"]["b"]),
                      l1["tg"]["W"], l1["gg"]["W"], l1["rt"]["W"])
    t1sc = t1.reshape(3 * NC1, NPAD, CW)
    s1a = _msg(t1sc, ei_all, zeros_hbm, 0, 2 * NC1, 0, NC1)
    s1b = _msg(t1sc, ei_all, zeros_hbm, 2 * NC1, NC1, 0, NC1)
    t2g = _tc_comb1g(s1a.reshape(2 * NC1, NPAD // 4, 128), dis,
                     r1(l1["tg"]["b"]), r1(l1["gg"]["b"]),
                     r1(l1["ln_gene"]["g"]), r1(l1["ln_gene"]["b"]),
                     l2["gg"]["W"], l2["rt"]["W"])
    s2g = _msg(t2g.reshape(2 * NC2, NPAD, CW), ei_all, zeros_hbm,
               NC2, 2 * NC2, NC2, NC2)
    t2d = _tc_comb1d(s1b.reshape(NC1, NPAD // 4, 128), dis,
                     r1(l1["rt"]["b"]),
                     r1(l1["ln_drug"]["g"]), r1(l1["ln_drug"]["b"]),
                     l2["tg"]["W"])
    s2d = _msg(t2d.reshape(NC2, NPAD, CW), ei_all, zeros_hbm,
               0, NC2, 0, NC2)
    drug_out, gene_out = _tc_comb2(
        s2d.reshape(NC2, NPAD // 4, 128),
        s2g.reshape(2 * NC2, NPAD // 4, 128), dis,
        r1(l2["tg"]["b"]), r1(l2["gg"]["b"]), r1(l2["rt"]["b"]),
        r1(l2["ln_gene"]["g"]), r1(l2["ln_gene"]["b"]),
        r1(l2["ln_drug"]["g"]), r1(l2["ln_drug"]["b"]))
    return drug_out[:N], gene_out[:N]
